# baseline (speedup 1.0000x reference)
import functools

import numpy as np
import jax
import jax.numpy as jnp
from jax import lax
from jax.experimental import pallas as pl
from jax.experimental.pallas import tpu as pltpu
from jax.experimental.pallas import tpu_sc as plsc

_NUM_FIELDS = 26
_FIELD_DIM = 40000
_BATCH = 16384
_EMB = 16
_TOTAL = _BATCH * _NUM_FIELDS
_NW = 32
_PER_W = _TOTAL // _NW
_IDXW = 128
_BLKS = 8
_NCHUNK = _PER_W // (_BLKS * _IDXW)
_TOTAL_BLKS = _TOTAL // _IDXW
_W_BLKS = _PER_W // _IDXW

_OFF_PAT = np.array(
    [(p % _NUM_FIELDS) * _FIELD_DIM for p in range(_PER_W)],
    dtype=np.int32)


def _sc_gather(x_flat, off_pat, table):
  mesh = plsc.VectorSubcoreMesh(core_axis_name="c", subcore_axis_name="s")

  @functools.partial(
      pl.kernel,
      mesh=mesh,
      compiler_params=pltpu.CompilerParams(use_tc_tiling_on_sc=False),
      out_type=jax.ShapeDtypeStruct((_TOTAL_BLKS, _IDXW, _EMB), jnp.float32),
      scratch_types=[
          pltpu.VMEM((_PER_W,), jnp.int32),
          pltpu.VMEM((_PER_W,), jnp.int32),
          pltpu.VMEM((_BLKS, _IDXW, _EMB), jnp.float32),
          pltpu.SemaphoreType.DMA,
      ],
  )
  def k(x_hbm, off_hbm, table_hbm, out_hbm, off_v, idx_v, rows_v, sem):
    wid = lax.axis_index("s") * 2 + lax.axis_index("c")
    base = wid * _PER_W

    pltpu.sync_copy(off_hbm, off_v)
    pltpu.sync_copy(x_hbm.at[pl.ds(base, _PER_W)], idx_v)

    def add_body(r, carry):
      for i in range(_IDXW // 16):
        s = pl.ds(r * _IDXW + i * 16, 16)
        idx_v[s] = idx_v[s] + off_v[s]
      return carry

    lax.fori_loop(0, _W_BLKS, add_body, 0)

    def chunk_body(c, carry):
      copies = [
          pltpu.async_copy(
              table_hbm.at[idx_v.at[pl.ds((c * _BLKS + j) * _IDXW, _IDXW)]],
              rows_v.at[j], sem)
          for j in range(_BLKS)
      ]
      for cp in copies:
        cp.wait()
      pltpu.sync_copy(
          rows_v, out_hbm.at[pl.ds(wid * _W_BLKS + c * _BLKS, _BLKS)])
      return carry

    lax.fori_loop(0, _NCHUNK, chunk_body, 0)

  return k(x_flat, off_pat, table)


def kernel(x, table):
  x_flat = x.reshape(_TOTAL).astype(jnp.int32)
  off = jnp.asarray(_OFF_PAT)
  out = _sc_gather(x_flat, off, table)
  return out.reshape(_BATCH, _NUM_FIELDS, _EMB)


# speedup vs baseline: 1.2970x; 1.2970x over previous
import functools

import jax
import jax.numpy as jnp
from jax import lax
from jax.experimental import pallas as pl
from jax.experimental.pallas import tpu as pltpu
from jax.experimental.pallas import tpu_sc as plsc

_NUM_FIELDS = 26
_FIELD_DIM = 40000
_BATCH = 16384
_EMB = 16
_TOTAL = _BATCH * _NUM_FIELDS
_NW = 32
_PER_W = _TOTAL // _NW
_IDXW = 128
_BLKS = 8
_CHUNK = _BLKS * _IDXW
_NCHUNK = _PER_W // _CHUNK
_TOTAL_BLKS = _TOTAL // _IDXW
_W_BLKS = _PER_W // _IDXW


def _sc_gather(x_fm, table):
  mesh = plsc.VectorSubcoreMesh(core_axis_name="c", subcore_axis_name="s")

  @functools.partial(
      pl.kernel,
      mesh=mesh,
      compiler_params=pltpu.CompilerParams(use_tc_tiling_on_sc=False),
      out_type=jax.ShapeDtypeStruct((_TOTAL_BLKS, _IDXW, _EMB), jnp.float32),
      scratch_types=[
          pltpu.VMEM((_PER_W,), jnp.int32),
          pltpu.VMEM((2, _BLKS, _IDXW, _EMB), jnp.float32),
          pltpu.SemaphoreType.DMA,
          pltpu.SemaphoreType.DMA,
          pltpu.SemaphoreType.DMA,
          pltpu.SemaphoreType.DMA,
      ],
  )
  def k(x_hbm, table_hbm, out_hbm, idx_v, rows_v, g0, g1, o0, o1):
    wid = lax.axis_index("s") * 2 + lax.axis_index("c")
    base = wid * _PER_W
    wblk = wid * _W_BLKS
    gsem = (g0, g1)
    osem = (o0, o1)

    pltpu.sync_copy(x_hbm.at[pl.ds(base, _PER_W)], idx_v)

    def out_slice(c):
      return out_hbm.at[pl.ds(wblk + c * _BLKS, _BLKS)]

    prev = None
    for c in range(_NCHUNK):
      slot = c & 1
      off = ((base + c * _CHUNK) >> 14) * _FIELD_DIM

      def add_body(i, carry, c=c, off=off):
        s = pl.ds(c * _CHUNK + i * 16, 16)
        idx_v[s] = idx_v[s] + off
        return carry

      lax.fori_loop(0, _CHUNK // 16, add_body, 0)

      if c >= 2:
        pltpu.make_async_copy(rows_v.at[slot], out_slice(c - 2),
                              osem[slot]).wait()
      hs = [
          pltpu.async_copy(
              table_hbm.at[idx_v.at[pl.ds(c * _CHUNK + j * _IDXW, _IDXW)]],
              rows_v.at[slot, j], gsem[slot])
          for j in range(_BLKS)
      ]
      if prev is not None:
        for h in prev:
          h.wait()
        pltpu.async_copy(rows_v.at[1 - slot], out_slice(c - 1),
                         osem[1 - slot])
      prev = hs

    last = _NCHUNK - 1
    for h in prev:
      h.wait()
    pltpu.async_copy(rows_v.at[last & 1], out_slice(last), osem[last & 1])
    pltpu.make_async_copy(rows_v.at[(last - 1) & 1], out_slice(last - 1),
                          osem[(last - 1) & 1]).wait()
    pltpu.make_async_copy(rows_v.at[last & 1], out_slice(last),
                          osem[last & 1]).wait()

  return k(x_fm, table)


def kernel(x, table):
  x_fm = x.T.reshape(_TOTAL).astype(jnp.int32)
  out = _sc_gather(x_fm, table)
  return out.reshape(_NUM_FIELDS, _BATCH, _EMB).transpose(1, 0, 2)
